# baseline (speedup 1.0000x reference)
import functools

import jax
import jax.numpy as jnp
from jax import lax
from jax.experimental import pallas as pl
from jax.experimental.pallas import tpu as pltpu
from jax.experimental.pallas import tpu_sc as plsc

EMBED = 128
TABLE_ROWS = 32
N_TOTAL = 3276800


def _fuse_kernel(w_ref, out_ref):
    r = lax.broadcasted_iota(jnp.int32, (TABLE_ROWS, 64), 0)
    c = lax.broadcasted_iota(jnp.int32, (TABLE_ROWS, 64), 1)
    sel = (c == r // 8) | (c == 8 + r % 8) | (c == 24 + r)
    onehot = sel.astype(jnp.float32)
    out_ref[...] = jnp.dot(onehot, w_ref[...], preferred_element_type=jnp.float32)


def _build_fused_table(level_emb, pos_emb, sin_table):
    w = jnp.zeros((64, EMBED), jnp.float32)
    w = w.at[0:4, 0:32].set(level_emb)
    w = w.at[8:17, 32:64].set(pos_emb)
    w = w.at[24:49, 64:128].set(sin_table)
    return pl.pallas_call(
        _fuse_kernel,
        out_shape=jax.ShapeDtypeStruct((TABLE_ROWS, EMBED), jnp.float32),
    )(w)


_CHUNK = 128
_GRP = 4


def _gather_body(n_grps, fused_hbm, pos_hbm, out_hbm, *scr):
    (idx_a, idx_b, rbuf, table_sh,
     sem_ia, sem_ib, sg0, sg1, sg2, sg3, so0, so1) = scr
    sem_g = (sg0, sg1, sg2, sg3)
    sem_o = (so0, so1)
    blk = _GRP * _CHUNK

    info = plsc.get_sparse_core_info()
    nc = info.num_cores
    sid = lax.axis_index("s")
    wid = sid * nc + lax.axis_index("c")
    per_w = n_grps * blk
    base = wid * per_w

    @pl.when(sid == 0)
    def _():
        pltpu.sync_copy(fused_hbm, table_sh)

    plsc.subcore_barrier()

    rbase = wid * n_grps * _GRP

    nw = info.num_cores * info.num_subcores

    def start_idx(j, idx_v, sem):
        off = (jnp.minimum(j, n_grps - 1) * nw + wid) * _GRP
        pltpu.make_async_copy(
            pos_hbm.at[pl.ds(off, _GRP)], idx_v.at[...], sem
        ).start()

    def wait_idx(idx_v, sem):
        pltpu.make_async_copy(
            pos_hbm.at[pl.ds(rbase, _GRP)], idx_v.at[...], sem
        ).wait()

    def slot(s):
        return rbuf.at[pl.ds(s * _CHUNK, _CHUNK)]

    def grp(g):
        return rbuf.at[pl.ds(g * 2 * _CHUNK, 2 * _CHUNK)]

    def wait_out(g):
        pltpu.make_async_copy(
            grp(g), out_hbm.at[pl.ds(base, 2 * _CHUNK)], sem_o[g]
        ).wait()

    def _gsrc(s):
        return table_sh if s % 2 == 0 else fused_hbm

    def start_gather(idx_v, c, s):
        pltpu.make_async_copy(_gsrc(s).at[idx_v.at[c]], slot(s), sem_g[s]).start()

    def wait_gather(idx_v, c, s):
        pltpu.make_async_copy(_gsrc(s).at[idx_v.at[c]], slot(s), sem_g[s]).wait()

    def start_out(j, g):
        off = (j * nw + wid) * blk + g * 2 * _CHUNK
        pltpu.make_async_copy(
            grp(g), out_hbm.at[pl.ds(off, 2 * _CHUNK)], sem_o[g]
        ).start()

    start_idx(0, idx_a, sem_ia)
    wait_idx(idx_a, sem_ia)
    start_idx(1, idx_b, sem_ib)
    start_gather(idx_a, 0, 0)

    def superstep(j, idx_v, idx_nv, sem_i, sem_in):
        start_gather(idx_v, 1, 1)
        wait_gather(idx_v, 0, 0)
        @pl.when(j >= 1)
        def _():
            wait_out(1)

        start_gather(idx_v, 2, 2)
        wait_gather(idx_v, 1, 1)
        start_out(j, 0)
        start_gather(idx_v, 3, 3)
        wait_gather(idx_v, 2, 2)
        wait_out(0)
        wait_idx(idx_nv, sem_in)
        start_gather(idx_nv, 0, 0)
        wait_gather(idx_v, 3, 3)
        start_out(j, 1)
        start_idx(j + 2, idx_v, sem_i)

    def pair(jj, _):
        superstep(2 * jj, idx_a, idx_b, sem_ia, sem_ib)
        superstep(2 * jj + 1, idx_b, idx_a, sem_ib, sem_ia)
        return 0

    lax.fori_loop(0, n_grps // 2, pair, 0)

    wait_gather(idx_a, 0, 0)
    wait_out(1)
    wait_idx(idx_b, sem_ib)


def _sc_gather(fused, positions):
    n = positions.shape[0]
    info = plsc.get_sparse_core_info()
    nw = info.num_cores * info.num_subcores
    blk = _GRP * _CHUNK
    n_grps = n // (nw * blk)
    assert n_grps * nw * blk == n and n_grps % 2 == 0
    mesh = plsc.VectorSubcoreMesh(core_axis_name="c", subcore_axis_name="s")
    grid_kernel = pl.kernel(
        functools.partial(_gather_body, n_grps),
        out_type=jax.ShapeDtypeStruct((n, EMBED), jnp.float32),
        mesh=mesh,
        scratch_types=[
            pltpu.VMEM((_GRP, _CHUNK), jnp.int32),
            pltpu.VMEM((_GRP, _CHUNK), jnp.int32),
            pltpu.VMEM((_GRP * _CHUNK, EMBED), jnp.float32),
            pltpu.VMEM_SHARED((TABLE_ROWS, EMBED), jnp.float32),
        ] + [pltpu.SemaphoreType.DMA] * 8,
    )
    return grid_kernel(fused, positions.reshape(n // _CHUNK, _CHUNK))


def kernel(positions, level_embedding, position_in_level_embedding, sinusoidal_table):
    positions = positions.astype(jnp.int32)
    fused = _build_fused_table(level_embedding, position_in_level_embedding,
                               sinusoidal_table)
    return _sc_gather(fused, positions)


# speedup vs baseline: 7.4292x; 7.4292x over previous
import functools

import jax
import jax.numpy as jnp
from jax import lax
from jax.experimental import pallas as pl
from jax.experimental.pallas import tpu as pltpu
from jax.experimental.pallas import tpu_sc as plsc

EMBED = 128
TABLE_ROWS = 32
N_TOTAL = 3276800


def _fuse_kernel(w_ref, out_ref):
    r = lax.broadcasted_iota(jnp.int32, (TABLE_ROWS, 64), 0)
    c = lax.broadcasted_iota(jnp.int32, (TABLE_ROWS, 64), 1)
    sel = (c == r // 8) | (c == 8 + r % 8) | (c == 24 + r)
    onehot = sel.astype(jnp.float32)
    out_ref[...] = jnp.dot(onehot, w_ref[...], preferred_element_type=jnp.float32)


def _build_fused_table(level_emb, pos_emb, sin_table):
    w = jnp.zeros((64, EMBED), jnp.float32)
    w = w.at[0:4, 0:32].set(level_emb)
    w = w.at[8:17, 32:64].set(pos_emb)
    w = w.at[24:49, 64:128].set(sin_table)
    return pl.pallas_call(
        _fuse_kernel,
        out_shape=jax.ShapeDtypeStruct((TABLE_ROWS, EMBED), jnp.float32),
    )(w)


_CHUNK = 128
_GRP = 4


def _gather_body(n_grps, fused_hbm, pos_hbm, out_hbm, *scr):
    (idx_a, idx_b, r0, r1, r2, r3, table_sh,
     sem_ia, sem_ib, sg0, sg1, sg2, sg3, so0, so1, so2, so3) = scr
    rows = (r0, r1, r2, r3)
    sem_g = (sg0, sg1, sg2, sg3)
    sem_o = (so0, so1, so2, so3)
    blk = _GRP * _CHUNK

    info = plsc.get_sparse_core_info()
    nc = info.num_cores
    sid = lax.axis_index("s")
    wid = sid * nc + lax.axis_index("c")
    per_w = n_grps * blk
    base = wid * per_w

    @pl.when(sid == 0)
    def _():
        pltpu.sync_copy(fused_hbm, table_sh)

    plsc.subcore_barrier()

    rbase = wid * n_grps * _GRP

    nw = info.num_cores * info.num_subcores

    def start_idx(j, idx_v, sem):
        off = (jnp.minimum(j, n_grps - 1) * nw + wid) * _GRP
        pltpu.make_async_copy(
            pos_hbm.at[pl.ds(off, _GRP)], idx_v.at[...], sem
        ).start()

    def wait_idx(idx_v, sem):
        pltpu.make_async_copy(
            pos_hbm.at[pl.ds(rbase, _GRP)], idx_v.at[...], sem
        ).wait()

    def wait_out(rows_v, sem):
        pltpu.make_async_copy(rows_v, out_hbm.at[pl.ds(base, _CHUNK)], sem).wait()

    def start_gather(idx_v, c, buf):
        pltpu.make_async_copy(
            table_sh.at[idx_v.at[c]], rows[buf], sem_g[buf]
        ).start()

    def wait_gather(idx_v, c, buf):
        pltpu.make_async_copy(
            table_sh.at[idx_v.at[c]], rows[buf], sem_g[buf]
        ).wait()

    def start_out(j, c, buf):
        off = (j * nw + wid) * blk + c * _CHUNK
        pltpu.make_async_copy(
            rows[buf], out_hbm.at[pl.ds(off, _CHUNK)], sem_o[buf]
        ).start()

    start_idx(0, idx_a, sem_ia)
    wait_idx(idx_a, sem_ia)
    start_idx(1, idx_b, sem_ib)
    start_gather(idx_a, 0, 0)
    start_gather(idx_a, 1, 1)

    def superstep(j, idx_v, idx_nv, sem_i, sem_in):
        for c in range(_GRP):
            i = j * _GRP + c
            nbuf = (c + 2) % _GRP

            @pl.when(i >= 2)
            def _():
                wait_out(rows[nbuf], sem_o[nbuf])

            if c < _GRP - 2:
                start_gather(idx_v, c + 2, nbuf)
            else:
                if c == _GRP - 2:
                    wait_idx(idx_nv, sem_in)
                start_gather(idx_nv, c + 2 - _GRP, nbuf)
            wait_gather(idx_v, c, c)
            start_out(j, c, c)
            if c == _GRP - 1:
                start_idx(j + 2, idx_v, sem_i)

    def pair(jj, _):
        superstep(2 * jj, idx_a, idx_b, sem_ia, sem_ib)
        superstep(2 * jj + 1, idx_b, idx_a, sem_ib, sem_ia)
        return 0

    lax.fori_loop(0, n_grps // 2, pair, 0)

    wait_gather(idx_a, 0, 0)
    wait_gather(idx_a, 1, 1)
    for c in range(2, _GRP):
        wait_out(rows[c], sem_o[c])
    wait_idx(idx_b, sem_ib)


def _sc_gather(fused, positions):
    n = positions.shape[0]
    info = plsc.get_sparse_core_info()
    nw = info.num_cores * info.num_subcores
    blk = _GRP * _CHUNK
    n_grps = n // (nw * blk)
    assert n_grps * nw * blk == n and n_grps % 2 == 0
    mesh = plsc.VectorSubcoreMesh(core_axis_name="c", subcore_axis_name="s")
    grid_kernel = pl.kernel(
        functools.partial(_gather_body, n_grps),
        out_type=jax.ShapeDtypeStruct((n, EMBED), jnp.float32),
        mesh=mesh,
        scratch_types=[
            pltpu.VMEM((_GRP, _CHUNK), jnp.int32),
            pltpu.VMEM((_GRP, _CHUNK), jnp.int32),
            pltpu.VMEM((_CHUNK, EMBED), jnp.float32),
            pltpu.VMEM((_CHUNK, EMBED), jnp.float32),
            pltpu.VMEM((_CHUNK, EMBED), jnp.float32),
            pltpu.VMEM((_CHUNK, EMBED), jnp.float32),
            pltpu.VMEM_SHARED((TABLE_ROWS, EMBED), jnp.float32),
        ] + [pltpu.SemaphoreType.DMA] * 10,
    )
    return grid_kernel(fused, positions.reshape(n // _CHUNK, _CHUNK))


def kernel(positions, level_embedding, position_in_level_embedding, sinusoidal_table):
    positions = positions.astype(jnp.int32)
    fused = _build_fused_table(level_embedding, position_in_level_embedding,
                               sinusoidal_table)
    return _sc_gather(fused, positions)
